# baseline (speedup 1.0000x reference)
import functools

import jax
import jax.numpy as jnp
from jax import lax
from jax.experimental import pallas as pl
from jax.experimental.pallas import tpu as pltpu
from jax.experimental.pallas import tpu_sc as plsc

D = 768
NUM_EXPERTS = 64
TOP_K = 2
ROUTER_DIM = 16
TOKENS = 32768

BLOCK = 4096

NC, NS, L = 2, 16, 16
NW = NC * NS
ROWS_W = TOKENS // NW


def _scores_block(x_ref, wq_ref, keys_ref, scores_ref, scores_t_ref):
    q = jax.lax.dot_general(
        x_ref[...], wq_ref[...], (((1,), (1,)), ((), ())),
        preferred_element_type=jnp.float32,
    )
    scores = jax.lax.dot_general(
        q, keys_ref[...], (((1,), (1,)), ((), ())),
        preferred_element_type=jnp.float32,
    )
    scores_ref[...] = scores
    scores_t_ref[...] = scores.T


def _tc_scores(x, W_query, keys):
    return pl.pallas_call(
        _scores_block,
        grid=(TOKENS // BLOCK,),
        in_specs=[
            pl.BlockSpec((BLOCK, D), lambda i: (i, 0)),
            pl.BlockSpec((ROUTER_DIM, D), lambda i: (0, 0)),
            pl.BlockSpec((NUM_EXPERTS, ROUTER_DIM), lambda i: (0, 0)),
        ],
        out_specs=(
            pl.BlockSpec((BLOCK, NUM_EXPERTS), lambda i: (i, 0)),
            pl.BlockSpec((NUM_EXPERTS, BLOCK), lambda i: (0, i)),
        ),
        out_shape=(
            jax.ShapeDtypeStruct((TOKENS, NUM_EXPERTS), jnp.float32),
            jax.ShapeDtypeStruct((NUM_EXPERTS, TOKENS), jnp.float32),
        ),
    )(x, W_query, keys)


@functools.partial(
    pl.kernel,
    mesh=plsc.VectorSubcoreMesh(core_axis_name="c", subcore_axis_name="s"),
    out_type=(
        jax.ShapeDtypeStruct((TOKENS,), jnp.int32),
        jax.ShapeDtypeStruct((TOKENS,), jnp.int32),
        jax.ShapeDtypeStruct((TOKENS,), jnp.float32),
        jax.ShapeDtypeStruct((TOKENS,), jnp.float32),
    ),
    scratch_types=[
        pltpu.VMEM((NUM_EXPERTS, ROWS_W), jnp.float32),
        pltpu.VMEM((ROWS_W,), jnp.int32),
        pltpu.VMEM((ROWS_W,), jnp.int32),
        pltpu.VMEM((ROWS_W,), jnp.float32),
        pltpu.VMEM((ROWS_W,), jnp.float32),
    ],
)
def _sc_route(scores_t_hbm, i1_hbm, i2_hbm, p1_hbm, p2_hbm,
              sc_t, sc_i1, sc_i2, sc_m1, sc_m2):
    wid = lax.axis_index("s") * NC + lax.axis_index("c")
    row0 = wid * ROWS_W
    pltpu.sync_copy(scores_t_hbm.at[:, pl.ds(row0, ROWS_W)], sc_t)

    ninf = jnp.full((L,), -jnp.inf, jnp.float32)
    zi = jnp.zeros((L,), jnp.int32)

    NCHAIN = 8
    span = NUM_EXPERTS // NCHAIN

    def group_body(g, carry):
        del carry
        sl = pl.ds(g * L, L)
        chains = []
        for c in range(NCHAIN):
            m1, m2 = ninf, ninf
            i1, i2 = zi, zi
            for e in range(c * span, (c + 1) * span):
                v = sc_t[e, sl]
                e_vec = jnp.full((L,), e, jnp.int32)
                gt1 = v > m1
                gt2 = v > m2
                i2 = jnp.where(gt1, i1, jnp.where(gt2, e_vec, i2))
                m2 = jnp.where(gt1, m1, jnp.where(gt2, v, m2))
                i1 = jnp.where(gt1, e_vec, i1)
                m1 = jnp.where(gt1, v, m1)
            chains.append((m1, i1, m2, i2))

        def merge(a, b):
            am1, ai1, am2, ai2 = a
            bm1, bi1, bm2, bi2 = b
            gt = bm1 > am1
            m1 = jnp.where(gt, bm1, am1)
            i1 = jnp.where(gt, bi1, ai1)
            a_over_b = am1 >= bm2
            m2g = jnp.where(a_over_b, am1, bm2)
            i2g = jnp.where(a_over_b, ai1, bi2)
            b_over_a = bm1 > am2
            m2n = jnp.where(b_over_a, bm1, am2)
            i2n = jnp.where(b_over_a, bi1, ai2)
            m2 = jnp.where(gt, m2g, m2n)
            i2 = jnp.where(gt, i2g, i2n)
            return (m1, i1, m2, i2)

        while len(chains) > 1:
            chains = [merge(chains[k], chains[k + 1])
                      for k in range(0, len(chains), 2)]
        m1, i1, m2, i2 = chains[0]
        ex = jnp.exp(m2 - m1)
        d = 1.0 + ex
        sc_i1[sl] = i1
        sc_i2[sl] = i2
        sc_m1[sl] = 1.0 / d
        sc_m2[sl] = ex / d
        return 0

    lax.fori_loop(0, ROWS_W // L, group_body, 0)

    pltpu.sync_copy(sc_i1, i1_hbm.at[pl.ds(row0, ROWS_W)])
    pltpu.sync_copy(sc_i2, i2_hbm.at[pl.ds(row0, ROWS_W)])
    pltpu.sync_copy(sc_m1, p1_hbm.at[pl.ds(row0, ROWS_W)])
    pltpu.sync_copy(sc_m2, p2_hbm.at[pl.ds(row0, ROWS_W)])


@jax.jit
def kernel(x, W_query, keys):
    scores, scores_t = _tc_scores(x, W_query, keys)
    i1, i2, p1, p2 = _sc_route(scores_t)
    return (jnp.stack([i1, i2], axis=1),
            jnp.stack([p1, p2], axis=1),
            scores)


# speedup vs baseline: 1.4916x; 1.4916x over previous
import jax
import jax.numpy as jnp
from jax.experimental import pallas as pl

D = 768
NUM_EXPERTS = 64
TOP_K = 2
ROUTER_DIM = 16
TOKENS = 32768

BLOCK = 4096


def _router_block(x_ref, wq_ref, keys_ref, scores_ref, idx_ref, probs_ref):
    q = jax.lax.dot_general(
        x_ref[...], wq_ref[...], (((1,), (1,)), ((), ())),
        preferred_element_type=jnp.float32,
    )
    scores = jax.lax.dot_general(
        q, keys_ref[...], (((1,), (1,)), ((), ())),
        preferred_element_type=jnp.float32,
    )
    scores_ref[...] = scores

    st = scores.T
    eidx = jax.lax.broadcasted_iota(jnp.int32, st.shape, 0)
    m1 = jnp.max(st, axis=0, keepdims=True)
    i1 = jnp.min(jnp.where(st == m1, eidx, NUM_EXPERTS),
                 axis=0, keepdims=True)
    masked = jnp.where(eidx == i1, -jnp.inf, st)
    m2 = jnp.max(masked, axis=0, keepdims=True)
    i2 = jnp.min(jnp.where(masked == m2, eidx, NUM_EXPERTS),
                 axis=0, keepdims=True)

    idx_ref[...] = jnp.concatenate([i1, i2], axis=0)
    e = jnp.exp(m2 - m1)
    denom = 1.0 + e
    probs_ref[...] = jnp.concatenate([1.0 / denom, e / denom], axis=0)


@jax.jit
def kernel(x, W_query, keys):
    scores, idx2, probs2 = pl.pallas_call(
        _router_block,
        grid=(TOKENS // BLOCK,),
        in_specs=[
            pl.BlockSpec((BLOCK, D), lambda i: (i, 0)),
            pl.BlockSpec((ROUTER_DIM, D), lambda i: (0, 0)),
            pl.BlockSpec((NUM_EXPERTS, ROUTER_DIM), lambda i: (0, 0)),
        ],
        out_specs=(
            pl.BlockSpec((BLOCK, NUM_EXPERTS), lambda i: (i, 0)),
            pl.BlockSpec((TOP_K, BLOCK), lambda i: (0, i)),
            pl.BlockSpec((TOP_K, BLOCK), lambda i: (0, i)),
        ),
        out_shape=(
            jax.ShapeDtypeStruct((TOKENS, NUM_EXPERTS), jnp.float32),
            jax.ShapeDtypeStruct((TOP_K, TOKENS), jnp.int32),
            jax.ShapeDtypeStruct((TOP_K, TOKENS), jnp.float32),
        ),
    )(x, W_query, keys)
    return idx2.T, probs2.T, scores
